# baseline (speedup 1.0000x reference)
import functools

import jax
import jax.numpy as jnp
from jax import lax
from jax.experimental import pallas as pl
from jax.experimental.pallas import tpu as pltpu
from jax.experimental.pallas import tpu_sc as plsc

NC = 2
NS = 16
NW = NC * NS
L = 16

HER_PROPORTION = 0.8
THRESHOLD = 0.05
TH_SQ = THRESHOLD * THRESHOLD


def _her_body(ach_hbm, des_hbm, rew_hbm, buf_hbm, noise_hbm, idx_hbm,
              goal_out, rew_out,
              idx_v, fut_v, ach_v, des_v, noise_v, rew_v, rewo_v, scr_v, gsem):
    B = ach_hbm.shape[0]
    TD = ach_hbm.shape[1]
    D = buf_hbm.shape[1]
    T = TD // D
    bpw = B // NW
    ngrp = bpw // L

    wid = lax.axis_index("s") * NC + lax.axis_index("c")
    base = wid * bpw

    pltpu.sync_copy(idx_hbm.at[pl.ds(base, bpw)], idx_v)
    gather = pltpu.async_copy(buf_hbm.at[idx_v], fut_v, gsem)
    pltpu.sync_copy(ach_hbm.at[pl.ds(base, bpw)], ach_v)
    pltpu.sync_copy(des_hbm.at[pl.ds(base, bpw)], des_v)
    pltpu.sync_copy(noise_hbm.at[pl.ds(base, bpw)], noise_v)
    for t in range(T):
        pltpu.sync_copy(rew_hbm.at[t, pl.ds(base, bpw)], rew_v.at[t])
    gather.wait()

    lane16 = lax.iota(jnp.int32, L) * L

    def group(g, carry):
        r0 = g * L
        nz = noise_v[pl.ds(r0, L)]
        cvec = nz < HER_PROPORTION
        for k in range(L):
            r = r0 + k
            cond = nz[k] < HER_PROPORTION
            for t in range(T):
                acc = jnp.zeros((L,), jnp.float32)
                for j in range(D // L):
                    col = t * D + j * L
                    a = ach_v[r, pl.ds(col, L)]
                    d = des_v[r, pl.ds(col, L)]
                    f = fut_v[r, pl.ds(j * L, L)]
                    gsel = jnp.where(cond, f, d)
                    des_v[r, pl.ds(col, L)] = gsel
                    diff = a - gsel
                    acc = acc + diff * diff
                plsc.store_scatter(scr_v, [lane16 + (t * L * L + k)], acc)
        for t in range(T):
            tot = scr_v[pl.ds(t * L * L, L)]
            for i in range(1, L):
                tot = tot + scr_v[pl.ds(t * L * L + i * L, L)]
            nr = -(tot >= TH_SQ).astype(jnp.float32)
            rewo_v[t, pl.ds(r0, L)] = jnp.where(cvec, nr,
                                                rew_v[t, pl.ds(r0, L)])
        return carry

    lax.fori_loop(0, ngrp, group, 0)

    pltpu.sync_copy(des_v, goal_out.at[pl.ds(base, bpw)])
    for t in range(T):
        pltpu.sync_copy(rewo_v.at[t], rew_out.at[t, pl.ds(base, bpw)])


def kernel(achieved_goal, desired_goal, reward, buffer_ag, her_noise, future_idx):
    B, T, D = achieved_goal.shape
    bpw = B // NW

    ach2d = achieved_goal.reshape(B, T * D)
    des2d = desired_goal.reshape(B, T * D)
    rew_t = reward.T
    idx32 = future_idx.astype(jnp.int32)

    mesh = plsc.VectorSubcoreMesh(core_axis_name="c", subcore_axis_name="s",
                                  num_cores=NC, num_subcores=NS)
    run = pl.kernel(
        _her_body,
        out_type=(
            jax.ShapeDtypeStruct((B, T * D), jnp.float32),
            jax.ShapeDtypeStruct((T, B), jnp.float32),
        ),
        mesh=mesh,
        compiler_params=pltpu.CompilerParams(needs_layout_passes=False,
                                             use_tc_tiling_on_sc=False),
        scratch_types=[
            pltpu.VMEM((bpw,), jnp.int32),
            pltpu.VMEM((bpw, D), jnp.float32),
            pltpu.VMEM((bpw, T * D), jnp.float32),
            pltpu.VMEM((bpw, T * D), jnp.float32),
            pltpu.VMEM((bpw,), jnp.float32),
            pltpu.VMEM((T, bpw), jnp.float32),
            pltpu.VMEM((T, bpw), jnp.float32),
            pltpu.VMEM((2 * 16 * 16,), jnp.float32),
            pltpu.SemaphoreType.DMA,
        ],
    )
    goal2d, rew2d = run(ach2d, des2d, rew_t, buffer_ag, her_noise, idx32)
    return goal2d.reshape(B, T, D), rew2d.T
